# Initial kernel scaffold; baseline (speedup 1.0000x reference)
import functools

import jax
import jax.numpy as jnp
from jax import lax
from jax.experimental import pallas as pl
from jax.experimental.pallas import tpu as pltpu
from jax.experimental.pallas import tpu_sc as plsc

V = 6
C = 64
FH = 32
FW = 88
BH = 128
BW = 128
NQ = BH * BW
IMG_H = 512.0
IMG_W = 1408.0
EPS = 1e-5

NC = 2
NS = 16
L = 16
NW = NC * NS
QPT = NQ // NW
NVR = QPT // L
G = 128
CAPP = V * QPT + G


def _sc_body(coef_hbm, table_hbm, s_hbm, m_hbm, sq_hbm,
             coef_v, mbuf, sq_v,
             i00, i01, i10, i11, qloc, w00, w01, w10, w11,
             acc, b00, b01, b10, b11, sem):
    cid = lax.axis_index("c")
    sid = lax.axis_index("s")
    wid = sid * NC + cid
    base_n = wid * QPT

    pltpu.sync_copy(coef_hbm, coef_v)

    zeros = jnp.zeros((L,), jnp.float32)
    izeros = jnp.zeros((L,), jnp.int32)

    def zacc(z, carry):
        acc[pl.ds(z * L, L)] = zeros
        return carry
    lax.fori_loop(0, QPT * C // L, zacc, 0)

    def zsq(z, carry):
        sq_v[pl.ds(z * L, L)] = zeros
        return carry
    lax.fori_loop(0, QPT // L, zsq, 0)

    lane = lax.iota(jnp.int32, L)
    lanef = lane.astype(jnp.float32)

    off = jnp.int32(0)
    for v in range(V):
        m00 = coef_v[v, 0]
        m01 = coef_v[v, 1]
        m02 = coef_v[v, 2]
        m03 = coef_v[v, 3]
        m10 = coef_v[v, 4]
        m11 = coef_v[v, 5]
        m12 = coef_v[v, 6]
        m13 = coef_v[v, 7]
        m20 = coef_v[v, 8]
        m21 = coef_v[v, 9]
        m22 = coef_v[v, 10]
        m23 = coef_v[v, 11]

        def geo(i16, off, v=v, m00=m00, m01=m01, m02=m02, m03=m03,
                m10=m10, m11=m11, m12=m12, m13=m13,
                m20=m20, m21=m21, m22=m22, m23=m23):
            i0 = i16 * L
            n0 = base_n + i0
            col0 = lax.rem(n0, BW)
            row0 = lax.div(n0, BW)
            colf = jnp.full((L,), col0, jnp.int32).astype(jnp.float32) + lanef
            rowf = jnp.full((L,), row0, jnp.int32).astype(jnp.float32)
            xg = (colf + 0.5) * (1.0 / BW)
            yg = (rowf + 0.5) * (1.0 / BH)
            x = xg * 102.4 + (-51.2)
            y = yg * 102.4 + (-51.2)
            z = x * 8.0 + (-5.0)
            p0 = m00 * x + m01 * y + m02 * z + m03
            p1 = m10 * x + m11 * y + m12 * z + m13
            p2 = m20 * x + m21 * y + m22 * z + m23
            dc = jnp.maximum(p2, EPS)
            xn = (p0 / dc) / IMG_W
            yn = (p1 / dc) / IMG_H
            valid = ((p2 > EPS) & (xn > 0.0) & (xn < 1.0)
                     & (yn > 0.0) & (yn < 1.0))
            gx = xn * FW - 0.5
            gy = yn * FH - 0.5
            xi = jnp.minimum(
                jnp.clip(gx, 0.0, FW - 1.0).astype(jnp.int32), FW - 2)
            yi = jnp.minimum(
                jnp.clip(gy, 0.0, FH - 1.0).astype(jnp.int32), FH - 2)
            wx = jnp.clip(gx - xi.astype(jnp.float32), 0.0, 1.0)
            wy = jnp.clip(gy - yi.astype(jnp.float32), 0.0, 1.0)
            mf = valid.astype(jnp.float32)
            r = yi * FW + xi + (v * FH * FW)
            ax = 1.0 - wx
            ay = 1.0 - wy
            cw00 = ax * ay * mf
            cw01 = wx * ay * mf
            cw10 = ax * wy * mf
            cw11 = wx * wy * mf
            sq_v[pl.ds(i0, L)] = sq_v[pl.ds(i0, L)] + mf
            mbuf[v, pl.ds(i0, L)] = mf
            qv = jnp.full((L,), i0, jnp.int32) + lane
            plsc.store_compressed(i00.at[pl.ds(off, L)], r, mask=valid)
            plsc.store_compressed(i01.at[pl.ds(off, L)], r + 1, mask=valid)
            plsc.store_compressed(i10.at[pl.ds(off, L)], r + FW, mask=valid)
            plsc.store_compressed(i11.at[pl.ds(off, L)], r + FW + 1,
                                  mask=valid)
            plsc.store_compressed(qloc.at[pl.ds(off, L)], qv, mask=valid)
            plsc.store_compressed(w00.at[pl.ds(off, L)], cw00, mask=valid)
            plsc.store_compressed(w01.at[pl.ds(off, L)], cw01, mask=valid)
            plsc.store_compressed(w10.at[pl.ds(off, L)], cw10, mask=valid)
            plsc.store_compressed(w11.at[pl.ds(off, L)], cw11, mask=valid)
            cnt = jnp.max(plsc.all_reduce_population_count(valid))
            return off + cnt

        off = lax.fori_loop(0, NVR, geo, off)

    for j in range(G // L):
        pos = off + j * L
        i00[pl.ds(pos, L)] = izeros
        i01[pl.ds(pos, L)] = izeros
        i10[pl.ds(pos, L)] = izeros
        i11[pl.ds(pos, L)] = izeros
        qloc[pl.ds(pos, L)] = izeros
        w00[pl.ds(pos, L)] = zeros
        w01[pl.ds(pos, L)] = zeros
        w10[pl.ds(pos, L)] = zeros
        w11[pl.ds(pos, L)] = zeros

    nch = lax.div(off + (G - 1), G)

    def chunk(ci, carry):
        st = ci * G
        cp0 = pltpu.async_copy(table_hbm.at[i00.at[pl.ds(st, G)]], b00, sem)
        cp1 = pltpu.async_copy(table_hbm.at[i01.at[pl.ds(st, G)]], b01, sem)
        cp2 = pltpu.async_copy(table_hbm.at[i10.at[pl.ds(st, G)]], b10, sem)
        cp3 = pltpu.async_copy(table_hbm.at[i11.at[pl.ds(st, G)]], b11, sem)
        cp0.wait()
        cp1.wait()
        cp2.wait()
        cp3.wait()

        def ent(e, c2):
            eidx = jnp.full((L,), st + e, jnp.int32)
            qs = plsc.load_gather(qloc, [eidx])
            s00 = plsc.load_gather(w00, [eidx])
            s01 = plsc.load_gather(w01, [eidx])
            s10 = plsc.load_gather(w10, [eidx])
            s11 = plsc.load_gather(w11, [eidx])
            qbase = qs * C
            for k in range(C // L):
                r00 = b00[e, pl.ds(k * L, L)]
                r01 = b01[e, pl.ds(k * L, L)]
                r10 = b10[e, pl.ds(k * L, L)]
                r11 = b11[e, pl.ds(k * L, L)]
                contrib = s00 * r00 + s01 * r01 + s10 * r10 + s11 * r11
                plsc.addupdate_scatter(acc, [qbase + (k * L) + lane], contrib)
            return c2
        lax.fori_loop(0, G, ent, 0)
        return carry

    lax.fori_loop(0, nch, chunk, 0)

    pltpu.sync_copy(acc, s_hbm.at[pl.ds(base_n * C, QPT * C)])
    pltpu.sync_copy(sq_v, sq_hbm.at[pl.ds(base_n, QPT)])
    pltpu.sync_copy(mbuf, m_hbm.at[:, pl.ds(base_n, QPT)])


_sc_kernel = functools.partial(
    pl.kernel,
    out_type=[
        jax.ShapeDtypeStruct((NQ * C,), jnp.float32),
        jax.ShapeDtypeStruct((V, NQ), jnp.float32),
        jax.ShapeDtypeStruct((NQ,), jnp.float32),
    ],
    mesh=plsc.VectorSubcoreMesh(core_axis_name="c", subcore_axis_name="s"),
    scratch_types=[
        pltpu.VMEM((V, 12, L), jnp.float32),
        pltpu.VMEM((V, QPT), jnp.float32),
        pltpu.VMEM((QPT,), jnp.float32),
        pltpu.VMEM((CAPP,), jnp.int32),
        pltpu.VMEM((CAPP,), jnp.int32),
        pltpu.VMEM((CAPP,), jnp.int32),
        pltpu.VMEM((CAPP,), jnp.int32),
        pltpu.VMEM((CAPP,), jnp.int32),
        pltpu.VMEM((CAPP,), jnp.float32),
        pltpu.VMEM((CAPP,), jnp.float32),
        pltpu.VMEM((CAPP,), jnp.float32),
        pltpu.VMEM((CAPP,), jnp.float32),
        pltpu.VMEM((QPT * C,), jnp.float32),
        pltpu.VMEM((G, C), jnp.float32),
        pltpu.VMEM((G, C), jnp.float32),
        pltpu.VMEM((G, C), jnp.float32),
        pltpu.VMEM((G, C), jnp.float32),
        pltpu.SemaphoreType.DMA,
    ],
)(_sc_body)


BQ = 2048


def _tc_body(bev_ref, s_ref, m_ref, sq_ref, ve_ref, wv_ref, wp_ref, wo_ref,
             bv_ref, bp_ref, bo_ref, out_ref):
    wc = jnp.dot(wv_ref[...],
                 jnp.dot(wp_ref[...], wo_ref[...],
                         preferred_element_type=jnp.float32),
                 preferred_element_type=jnp.float32)
    d1 = jnp.dot(jnp.dot(bv_ref[...], wp_ref[...],
                         preferred_element_type=jnp.float32) + bp_ref[...],
                 wo_ref[...], preferred_element_type=jnp.float32)
    e = lax.dot_general(m_ref[...], ve_ref[...], (((0,), (0,)), ((), ())),
                        preferred_element_type=jnp.float32)
    pre = s_ref[...] + e
    sq = sq_ref[...]
    invc = 1.0 / jnp.maximum(sq, 1.0)
    out_ref[...] = (bev_ref[...]
                    + jnp.dot(pre, wc, preferred_element_type=jnp.float32)
                    * invc
                    + (sq * invc) * d1 + bo_ref[...])


def _tc_call(bev, s, m, sq, ve, wv, wp, wo, bv, bp, bo):
    grid = (NQ // BQ,)
    return pl.pallas_call(
        _tc_body,
        grid=grid,
        in_specs=[
            pl.BlockSpec((BQ, C), lambda q: (q, 0)),
            pl.BlockSpec((BQ, C), lambda q: (q, 0)),
            pl.BlockSpec((V, BQ), lambda q: (0, q)),
            pl.BlockSpec((BQ, 1), lambda q: (q, 0)),
            pl.BlockSpec((V, C), lambda q: (0, 0)),
            pl.BlockSpec((C, C), lambda q: (0, 0)),
            pl.BlockSpec((C, C), lambda q: (0, 0)),
            pl.BlockSpec((C, C), lambda q: (0, 0)),
            pl.BlockSpec((1, C), lambda q: (0, 0)),
            pl.BlockSpec((1, C), lambda q: (0, 0)),
            pl.BlockSpec((1, C), lambda q: (0, 0)),
        ],
        out_specs=pl.BlockSpec((BQ, C), lambda q: (q, 0)),
        out_shape=jax.ShapeDtypeStruct((NQ, C), jnp.float32),
    )(bev, s, m, sq, ve, wv, wp, wo, bv, bp, bo)


def kernel(feat, lidar2img, bev_table, view_embeds, W_attn, b_attn,
           W_val, b_val, W_proj, b_proj, W_out, b_out):
    del W_attn, b_attn
    table = feat.transpose(0, 2, 3, 1).reshape(V * FH * FW, C)
    coef = lidar2img[:, :3, :].reshape(V, 12)
    coef16 = jnp.broadcast_to(coef[:, :, None], (V, 12, L))
    coef16 = jnp.asarray(coef16, jnp.float32)

    s_flat, m, sq = _sc_kernel(coef16, table)

    out = _tc_call(
        bev_table, s_flat.reshape(NQ, C), m, sq.reshape(NQ, 1),
        view_embeds, W_val, W_proj, W_out,
        b_val.reshape(1, C), b_proj.reshape(1, C), b_out.reshape(1, C))
    return out


# speedup vs baseline: 174.9014x; 174.9014x over previous
import functools

import jax
import jax.numpy as jnp
from jax import lax
from jax.experimental import pallas as pl
from jax.experimental.pallas import tpu as pltpu
from jax.experimental.pallas import tpu_sc as plsc

V = 6
C = 64
FH = 32
FW = 88
BH = 128
BW = 128
NQ = BH * BW
IMG_H = 512.0
IMG_W = 1408.0
EPS = 1e-5

NC = 2
NS = 16
L = 16
NW = NC * NS
QPT = NQ // NW
NVR = QPT // L
CAPV = QPT + L
NP = V * FH * FW


def _lane_gather(x, idx):
    dn = lax.GatherDimensionNumbers(offset_dims=(), collapsed_slice_dims=(0,),
                                    start_index_map=(0,))
    return lax.gather(x, idx[:, None], dn, (1,),
                      mode=lax.GatherScatterMode.PROMISE_IN_BOUNDS)


def _bf16r(x):
    u = lax.bitcast_convert_type(x, jnp.int32)
    r = jnp.where((u & 0x10000) != 0, 1, 0)
    u = u + 0x7FFF + r
    u = u & jnp.int32(-65536)
    return lax.bitcast_convert_type(u, jnp.float32)


def _sc_body(coef_hbm, table_hbm, s_hbm, sq_hbm,
             coef_v, sq_v,
             ip0, ip1, qloc, w00, w01, w10, w11,
             acc, bp0, bp1, sem):
    cid = lax.axis_index("c")
    sid = lax.axis_index("s")
    wid = sid * NC + cid
    base_n = wid * QPT

    pltpu.sync_copy(coef_hbm, coef_v)

    zeros = jnp.zeros((L,), jnp.float32)

    def zsq(z, carry):
        sq_v[pl.ds(z * L, L)] = zeros
        return carry
    lax.fori_loop(0, QPT // L, zsq, 0)

    lane = lax.iota(jnp.int32, L)
    lanef = lane.astype(jnp.float32)

    for v in range(V):
        m00 = _bf16r(coef_v[v, 0])
        m01 = _bf16r(coef_v[v, 1])
        m02 = _bf16r(coef_v[v, 2])
        m03 = _bf16r(coef_v[v, 3])
        m10 = _bf16r(coef_v[v, 4])
        m11 = _bf16r(coef_v[v, 5])
        m12 = _bf16r(coef_v[v, 6])
        m13 = _bf16r(coef_v[v, 7])
        m20 = _bf16r(coef_v[v, 8])
        m21 = _bf16r(coef_v[v, 9])
        m22 = _bf16r(coef_v[v, 10])
        m23 = _bf16r(coef_v[v, 11])

        def zacc(z, carry):
            acc[pl.ds(z * L, L)] = zeros
            return carry
        lax.fori_loop(0, QPT * C // L, zacc, 0)

        def geo(i16, off, v=v, m00=m00, m01=m01, m02=m02, m03=m03,
                m10=m10, m11=m11, m12=m12, m13=m13,
                m20=m20, m21=m21, m22=m22, m23=m23):
            i0 = i16 * L
            n0 = base_n + i0
            col0 = lax.rem(n0, BW)
            row0 = lax.div(n0, BW)
            colf = jnp.full((L,), col0, jnp.int32).astype(jnp.float32) + lanef
            rowf = jnp.full((L,), row0, jnp.int32).astype(jnp.float32)
            xg = (colf + 0.5) * (1.0 / BW)
            yg = (rowf + 0.5) * (1.0 / BH)
            xf = xg * 102.4 + (-51.2)
            yf = yg * 102.4 + (-51.2)
            zf = xf * 8.0 + (-5.0)
            x = _bf16r(xf)
            y = _bf16r(yf)
            z = _bf16r(zf)
            p0 = m00 * x + m01 * y + m02 * z + m03
            p1 = m10 * x + m11 * y + m12 * z + m13
            p2 = m20 * x + m21 * y + m22 * z + m23
            dc = jnp.maximum(p2, EPS)
            xn = (p0 / dc) / IMG_W
            yn = (p1 / dc) / IMG_H
            valid = ((p2 > EPS) & (xn > 0.0) & (xn < 1.0)
                     & (yn > 0.0) & (yn < 1.0))
            gx = xn * FW - 0.5
            gy = yn * FH - 0.5
            xi = jnp.minimum(
                jnp.clip(gx, 0.0, FW - 1.0).astype(jnp.int32), FW - 2)
            yi = jnp.minimum(
                jnp.clip(gy, 0.0, FH - 1.0).astype(jnp.int32), FH - 2)
            wx = jnp.clip(gx - xi.astype(jnp.float32), 0.0, 1.0)
            wy = jnp.clip(gy - yi.astype(jnp.float32), 0.0, 1.0)
            mf = jnp.where(valid, 1.0, 0.0)
            r = yi * FW + xi + (v * FH * FW)
            ax = 1.0 - wx
            ay = 1.0 - wy
            cw00 = ax * ay * mf
            cw01 = wx * ay * mf
            cw10 = ax * wy * mf
            cw11 = wx * wy * mf
            sq_v[pl.ds(i0, L)] = sq_v[pl.ds(i0, L)] + mf
            qv = jnp.full((L,), i0, jnp.int32) + lane
            vi = jnp.where(valid, 1, 0)
            csum = vi
            for sh in (1, 2, 4, 8):
                sidx = jnp.maximum(lane - sh, 0)
                moved = _lane_gather(csum, sidx)
                csum = csum + jnp.where(lane >= sh, moved, 0)
            nvalid = csum[L - 1]

            @pl.when(nvalid > 0)
            def _():
                ip0[pl.ds(off, L)] = jnp.where(valid, r, 0)
                ip1[pl.ds(off, L)] = jnp.where(valid, r + FW, 0)
                qloc[pl.ds(off, L)] = qv
                w00[pl.ds(off, L)] = cw00
                w01[pl.ds(off, L)] = cw01
                w10[pl.ds(off, L)] = cw10
                w11[pl.ds(off, L)] = cw11

            return jnp.where(nvalid > 0, off + L, off)

        off = lax.fori_loop(0, NVR, geo, jnp.int32(0))

        ngrp = lax.div(off, L)

        def grp(g, carry):
            eb = g * L
            idx16a = ip0[pl.ds(eb, L)]
            idx16b = ip1[pl.ds(eb, L)]
            cp0 = pltpu.async_copy(table_hbm.at[idx16a], bp0, sem)
            cp1 = pltpu.async_copy(table_hbm.at[idx16b], bp1, sem)
            q16 = qloc[pl.ds(eb, L)]
            w00g = w00[pl.ds(eb, L)]
            w01g = w01[pl.ds(eb, L)]
            w10g = w10[pl.ds(eb, L)]
            w11g = w11[pl.ds(eb, L)]
            abase = q16[0] * C
            cp0.wait()
            cp1.wait()
            for li in range(L):
                bidx = jnp.full((L,), li, jnp.int32)
                s00 = _lane_gather(w00g, bidx)
                s01 = _lane_gather(w01g, bidx)
                s10 = _lane_gather(w10g, bidx)
                s11 = _lane_gather(w11g, bidx)
                for k in range(C // L):
                    r00 = bp0[li, pl.ds(k * L, L)]
                    r01 = bp0[li, pl.ds(C + k * L, L)]
                    r10 = bp1[li, pl.ds(k * L, L)]
                    r11 = bp1[li, pl.ds(C + k * L, L)]
                    contrib = s00 * r00 + s01 * r01 + s10 * r10 + s11 * r11
                    sl = abase + li * C + k * L
                    acc[pl.ds(sl, L)] = acc[pl.ds(sl, L)] + contrib
            return carry

        lax.fori_loop(0, ngrp, grp, 0)

        pltpu.sync_copy(
            acc, s_hbm.at[pl.ds(v * NQ * C + base_n * C, QPT * C)])

    pltpu.sync_copy(sq_v, sq_hbm.at[pl.ds(base_n, QPT)])


_sc_kernel = functools.partial(
    pl.kernel,
    out_type=[
        jax.ShapeDtypeStruct((V * NQ * C,), jnp.float32),
        jax.ShapeDtypeStruct((NQ,), jnp.float32),
    ],
    mesh=plsc.VectorSubcoreMesh(core_axis_name="c", subcore_axis_name="s"),
    scratch_types=[
        pltpu.VMEM((V, 12, L), jnp.float32),
        pltpu.VMEM((QPT,), jnp.float32),
        pltpu.VMEM((CAPV,), jnp.int32),
        pltpu.VMEM((CAPV,), jnp.int32),
        pltpu.VMEM((CAPV,), jnp.int32),
        pltpu.VMEM((CAPV,), jnp.float32),
        pltpu.VMEM((CAPV,), jnp.float32),
        pltpu.VMEM((CAPV,), jnp.float32),
        pltpu.VMEM((CAPV,), jnp.float32),
        pltpu.VMEM((QPT * C,), jnp.float32),
        pltpu.VMEM((L, 2 * C), jnp.float32),
        pltpu.VMEM((L, 2 * C), jnp.float32),
        pltpu.SemaphoreType.DMA,
    ],
)(_sc_body)


def _ta_body(featp_ref, ve_ref, wv_ref, bv_ref, out_ref):
    value = featp_ref[0] + ve_ref[0]
    out_ref[0, :, :] = (jnp.dot(value, wv_ref[...],
                                preferred_element_type=jnp.float32)
                        + bv_ref[...])


def _ta_call(featp, ve, wv, bv):
    return pl.pallas_call(
        _ta_body,
        grid=(V,),
        in_specs=[
            pl.BlockSpec((1, FH * FW, C), lambda v: (v, 0, 0)),
            pl.BlockSpec((1, 1, C), lambda v: (v, 0, 0)),
            pl.BlockSpec((C, C), lambda v: (0, 0)),
            pl.BlockSpec((1, C), lambda v: (0, 0)),
        ],
        out_specs=pl.BlockSpec((1, FH * FW, C), lambda v: (v, 0, 0)),
        out_shape=jax.ShapeDtypeStruct((V, FH * FW, C), jnp.float32),
    )(featp, ve, wv, bv)


BQ = 2048


def _tc_body(bev_ref, sv_ref, sq_ref, wp_ref, wo_ref,
             bp_ref, bo_ref, out_ref):
    acc = jnp.dot(sv_ref[0], wp_ref[...],
                  preferred_element_type=jnp.float32)
    for v in range(1, V):
        acc = acc + jnp.dot(sv_ref[v], wp_ref[...],
                            preferred_element_type=jnp.float32)
    sq = sq_ref[...]
    cnt = jnp.maximum(sq, 1.0)
    slots = (acc + sq * bp_ref[...]) / cnt
    out_ref[...] = (bev_ref[...]
                    + jnp.dot(slots, wo_ref[...],
                              preferred_element_type=jnp.float32)
                    + bo_ref[...])


def _tc_call(bev, sv, sq, wp, wo, bp, bo):
    return pl.pallas_call(
        _tc_body,
        grid=(NQ // BQ,),
        in_specs=[
            pl.BlockSpec((BQ, C), lambda q: (q, 0)),
            pl.BlockSpec((V, BQ, C), lambda q: (0, q, 0)),
            pl.BlockSpec((BQ, 1), lambda q: (q, 0)),
            pl.BlockSpec((C, C), lambda q: (0, 0)),
            pl.BlockSpec((C, C), lambda q: (0, 0)),
            pl.BlockSpec((1, C), lambda q: (0, 0)),
            pl.BlockSpec((1, C), lambda q: (0, 0)),
        ],
        out_specs=pl.BlockSpec((BQ, C), lambda q: (q, 0)),
        out_shape=jax.ShapeDtypeStruct((NQ, C), jnp.float32),
    )(bev, sv, sq, wp, wo, bp, bo)


def kernel(feat, lidar2img, bev_table, view_embeds, W_attn, b_attn,
           W_val, b_val, W_proj, b_proj, W_out, b_out):
    del W_attn, b_attn
    featp = feat.transpose(0, 2, 3, 1)
    val = _ta_call(featp.reshape(V, FH * FW, C),
                   view_embeds.reshape(V, 1, C), W_val, b_val.reshape(1, C))
    v2 = val.reshape(NP, C)
    v2n = jnp.concatenate([v2[1:], v2[:1]], axis=0)
    table = jnp.concatenate([v2, v2n], axis=1)

    coef = lidar2img[:, :3, :].reshape(V, 12)
    coef16 = jnp.broadcast_to(coef[:, :, None], (V, 12, L))
    coef16 = jnp.asarray(coef16, jnp.float32)

    sv_flat, sq = _sc_kernel(coef16, table)

    out = _tc_call(
        bev_table, sv_flat.reshape(V, NQ, C), sq.reshape(NQ, 1),
        W_proj, W_out, b_proj.reshape(1, C), b_out.reshape(1, C))
    return out


# speedup vs baseline: 185.9162x; 1.0630x over previous
import functools

import jax
import jax.numpy as jnp
from jax import lax
from jax.experimental import pallas as pl
from jax.experimental.pallas import tpu as pltpu
from jax.experimental.pallas import tpu_sc as plsc

V = 6
C = 64
FH = 32
FW = 88
BH = 128
BW = 128
NQ = BH * BW
IMG_H = 512.0
IMG_W = 1408.0
EPS = 1e-5

NC = 2
NS = 16
L = 16
NW = NC * NS
QPT = NQ // NW
NVR = QPT // L
CAPV = QPT + L
NP = V * FH * FW


def _lane_gather(x, idx):
    dn = lax.GatherDimensionNumbers(offset_dims=(), collapsed_slice_dims=(0,),
                                    start_index_map=(0,))
    return lax.gather(x, idx[:, None], dn, (1,),
                      mode=lax.GatherScatterMode.PROMISE_IN_BOUNDS)


def _bf16r(x):
    u = lax.bitcast_convert_type(x, jnp.int32)
    r = jnp.where((u & 0x10000) != 0, 1, 0)
    u = u + 0x7FFF + r
    u = u & jnp.int32(-65536)
    return lax.bitcast_convert_type(u, jnp.float32)


def _sc_body(coef_hbm, table_hbm, s_hbm, sq_hbm,
             coef_v, sq_v,
             ip0, ip1, qloc, w00, w01, w10, w11,
             acc, bp0, bp1, sem):
    cid = lax.axis_index("c")
    sid = lax.axis_index("s")
    wid = sid * NC + cid
    base_n = wid * QPT

    pltpu.sync_copy(coef_hbm, coef_v)

    zeros = jnp.zeros((L,), jnp.float32)

    def zsq(z, carry):
        sq_v[pl.ds(z * L, L)] = zeros
        return carry
    lax.fori_loop(0, QPT // L, zsq, 0)

    lane = lax.iota(jnp.int32, L)
    lanef = lane.astype(jnp.float32)

    for v in range(V):
        m00 = _bf16r(coef_v[v, 0])
        m01 = _bf16r(coef_v[v, 1])
        m02 = _bf16r(coef_v[v, 2])
        m03 = _bf16r(coef_v[v, 3])
        m10 = _bf16r(coef_v[v, 4])
        m11 = _bf16r(coef_v[v, 5])
        m12 = _bf16r(coef_v[v, 6])
        m13 = _bf16r(coef_v[v, 7])
        m20 = _bf16r(coef_v[v, 8])
        m21 = _bf16r(coef_v[v, 9])
        m22 = _bf16r(coef_v[v, 10])
        m23 = _bf16r(coef_v[v, 11])

        def zacc(z, carry):
            for u in range(8):
                acc[pl.ds(z * (8 * L) + u * L, L)] = zeros
            return carry
        lax.fori_loop(0, QPT * C // (8 * L), zacc, 0)

        def geo(i16, off, v=v, m00=m00, m01=m01, m02=m02, m03=m03,
                m10=m10, m11=m11, m12=m12, m13=m13,
                m20=m20, m21=m21, m22=m22, m23=m23):
            i0 = i16 * L
            n0 = base_n + i0
            col0 = lax.rem(n0, BW)
            row0 = lax.div(n0, BW)
            colf = jnp.full((L,), col0, jnp.int32).astype(jnp.float32) + lanef
            rowf = jnp.full((L,), row0, jnp.int32).astype(jnp.float32)
            xg = (colf + 0.5) * (1.0 / BW)
            yg = (rowf + 0.5) * (1.0 / BH)
            xf = xg * 102.4 + (-51.2)
            yf = yg * 102.4 + (-51.2)
            zf = xf * 8.0 + (-5.0)
            x = _bf16r(xf)
            y = _bf16r(yf)
            z = _bf16r(zf)
            p0 = m00 * x + m01 * y + m02 * z + m03
            p1 = m10 * x + m11 * y + m12 * z + m13
            p2 = m20 * x + m21 * y + m22 * z + m23
            dc = jnp.maximum(p2, EPS)
            xn = (p0 / dc) / IMG_W
            yn = (p1 / dc) / IMG_H
            valid = ((p2 > EPS) & (xn > 0.0) & (xn < 1.0)
                     & (yn > 0.0) & (yn < 1.0))
            gx = xn * FW - 0.5
            gy = yn * FH - 0.5
            xi = jnp.minimum(
                jnp.clip(gx, 0.0, FW - 1.0).astype(jnp.int32), FW - 2)
            yi = jnp.minimum(
                jnp.clip(gy, 0.0, FH - 1.0).astype(jnp.int32), FH - 2)
            wx = jnp.clip(gx - xi.astype(jnp.float32), 0.0, 1.0)
            wy = jnp.clip(gy - yi.astype(jnp.float32), 0.0, 1.0)
            mf = jnp.where(valid, 1.0, 0.0)
            r = yi * FW + xi + (v * FH * FW)
            ax = 1.0 - wx
            ay = 1.0 - wy
            cw00 = ax * ay * mf
            cw01 = wx * ay * mf
            cw10 = ax * wy * mf
            cw11 = wx * wy * mf
            sq_v[pl.ds(i0, L)] = sq_v[pl.ds(i0, L)] + mf
            qv = jnp.full((L,), i0, jnp.int32) + lane
            vi = jnp.where(valid, 1, 0)
            csum = vi
            for sh in (1, 2, 4, 8):
                sidx = jnp.maximum(lane - sh, 0)
                moved = _lane_gather(csum, sidx)
                csum = csum + jnp.where(lane >= sh, moved, 0)
            nvalid = csum[L - 1]

            @pl.when(nvalid > 0)
            def _():
                ip0[pl.ds(off, L)] = jnp.where(valid, r, 0)
                ip1[pl.ds(off, L)] = jnp.where(valid, r + FW, 0)
                qloc[pl.ds(off, L)] = qv
                w00[pl.ds(off, L)] = cw00
                w01[pl.ds(off, L)] = cw01
                w10[pl.ds(off, L)] = cw10
                w11[pl.ds(off, L)] = cw11

            return jnp.where(nvalid > 0, off + L, off)

        off = lax.fori_loop(0, NVR, geo, jnp.int32(0))

        ngrp = lax.div(off, L)

        def grp(g, carry):
            eb = g * L
            idx16a = ip0[pl.ds(eb, L)]
            idx16b = ip1[pl.ds(eb, L)]
            cp0 = pltpu.async_copy(table_hbm.at[idx16a], bp0, sem)
            cp1 = pltpu.async_copy(table_hbm.at[idx16b], bp1, sem)
            q16 = qloc[pl.ds(eb, L)]
            w00g = w00[pl.ds(eb, L)]
            w01g = w01[pl.ds(eb, L)]
            w10g = w10[pl.ds(eb, L)]
            w11g = w11[pl.ds(eb, L)]
            abase = q16[0] * C
            cp0.wait()
            cp1.wait()
            for li in range(L):
                bidx = jnp.full((L,), li, jnp.int32)
                s00 = _lane_gather(w00g, bidx)
                s01 = _lane_gather(w01g, bidx)
                s10 = _lane_gather(w10g, bidx)
                s11 = _lane_gather(w11g, bidx)
                for k in range(C // L):
                    r00 = bp0[li, pl.ds(k * L, L)]
                    r01 = bp0[li, pl.ds(C + k * L, L)]
                    r10 = bp1[li, pl.ds(k * L, L)]
                    r11 = bp1[li, pl.ds(C + k * L, L)]
                    contrib = s00 * r00 + s01 * r01 + s10 * r10 + s11 * r11
                    sl = abase + li * C + k * L
                    acc[pl.ds(sl, L)] = acc[pl.ds(sl, L)] + contrib
            return carry

        lax.fori_loop(0, ngrp, grp, 0)

        pltpu.sync_copy(
            acc, s_hbm.at[pl.ds(v * NQ * C + base_n * C, QPT * C)])

    pltpu.sync_copy(sq_v, sq_hbm.at[pl.ds(base_n, QPT)])


_sc_kernel = functools.partial(
    pl.kernel,
    out_type=[
        jax.ShapeDtypeStruct((V * NQ * C,), jnp.float32),
        jax.ShapeDtypeStruct((NQ,), jnp.float32),
    ],
    mesh=plsc.VectorSubcoreMesh(core_axis_name="c", subcore_axis_name="s"),
    scratch_types=[
        pltpu.VMEM((V, 12, L), jnp.float32),
        pltpu.VMEM((QPT,), jnp.float32),
        pltpu.VMEM((CAPV,), jnp.int32),
        pltpu.VMEM((CAPV,), jnp.int32),
        pltpu.VMEM((CAPV,), jnp.int32),
        pltpu.VMEM((CAPV,), jnp.float32),
        pltpu.VMEM((CAPV,), jnp.float32),
        pltpu.VMEM((CAPV,), jnp.float32),
        pltpu.VMEM((CAPV,), jnp.float32),
        pltpu.VMEM((QPT * C,), jnp.float32),
        pltpu.VMEM((L, 2 * C), jnp.float32),
        pltpu.VMEM((L, 2 * C), jnp.float32),
        pltpu.SemaphoreType.DMA,
    ],
)(_sc_body)


def _ta_body(featp_ref, ve_ref, wv_ref, bv_ref, out_ref):
    value = featp_ref[0] + ve_ref[0]
    out_ref[0, :, :] = (jnp.dot(value, wv_ref[...],
                                preferred_element_type=jnp.float32)
                        + bv_ref[...])


def _ta_call(featp, ve, wv, bv):
    return pl.pallas_call(
        _ta_body,
        grid=(V,),
        in_specs=[
            pl.BlockSpec((1, FH * FW, C), lambda v: (v, 0, 0)),
            pl.BlockSpec((1, 1, C), lambda v: (v, 0, 0)),
            pl.BlockSpec((C, C), lambda v: (0, 0)),
            pl.BlockSpec((1, C), lambda v: (0, 0)),
        ],
        out_specs=pl.BlockSpec((1, FH * FW, C), lambda v: (v, 0, 0)),
        out_shape=jax.ShapeDtypeStruct((V, FH * FW, C), jnp.float32),
    )(featp, ve, wv, bv)


BQ = 2048


def _tc_body(bev_ref, sv_ref, sq_ref, wp_ref, wo_ref,
             bp_ref, bo_ref, out_ref):
    acc = jnp.dot(sv_ref[0], wp_ref[...],
                  preferred_element_type=jnp.float32)
    for v in range(1, V):
        acc = acc + jnp.dot(sv_ref[v], wp_ref[...],
                            preferred_element_type=jnp.float32)
    sq = sq_ref[...]
    cnt = jnp.maximum(sq, 1.0)
    slots = (acc + sq * bp_ref[...]) / cnt
    out_ref[...] = (bev_ref[...]
                    + jnp.dot(slots, wo_ref[...],
                              preferred_element_type=jnp.float32)
                    + bo_ref[...])


def _tc_call(bev, sv, sq, wp, wo, bp, bo):
    return pl.pallas_call(
        _tc_body,
        grid=(NQ // BQ,),
        in_specs=[
            pl.BlockSpec((BQ, C), lambda q: (q, 0)),
            pl.BlockSpec((V, BQ, C), lambda q: (0, q, 0)),
            pl.BlockSpec((BQ, 1), lambda q: (q, 0)),
            pl.BlockSpec((C, C), lambda q: (0, 0)),
            pl.BlockSpec((C, C), lambda q: (0, 0)),
            pl.BlockSpec((1, C), lambda q: (0, 0)),
            pl.BlockSpec((1, C), lambda q: (0, 0)),
        ],
        out_specs=pl.BlockSpec((BQ, C), lambda q: (q, 0)),
        out_shape=jax.ShapeDtypeStruct((NQ, C), jnp.float32),
    )(bev, sv, sq, wp, wo, bp, bo)


def kernel(feat, lidar2img, bev_table, view_embeds, W_attn, b_attn,
           W_val, b_val, W_proj, b_proj, W_out, b_out):
    del W_attn, b_attn
    featp = feat.transpose(0, 2, 3, 1)
    val = _ta_call(featp.reshape(V, FH * FW, C),
                   view_embeds.reshape(V, 1, C), W_val, b_val.reshape(1, C))
    v2 = val.reshape(NP, C)
    v2n = jnp.concatenate([v2[1:], v2[:1]], axis=0)
    table = jnp.concatenate([v2, v2n], axis=1)

    coef = lidar2img[:, :3, :].reshape(V, 12)
    coef16 = jnp.broadcast_to(coef[:, :, None], (V, 12, L))
    coef16 = jnp.asarray(coef16, jnp.float32)

    sv_flat, sq = _sc_kernel(coef16, table)

    out = _tc_call(
        bev_table, sv_flat.reshape(V, NQ, C), sq.reshape(NQ, 1),
        W_proj, W_out, b_proj.reshape(1, C), b_out.reshape(1, C))
    return out
